# baseline (speedup 1.0000x reference)
import jax
import jax.numpy as jnp
from jax.experimental import pallas as pl
from jax.experimental.pallas import tpu as pltpu

_B = 1024
_N = 1000
_NPAD = 1024
_M = 50
_K = 6
_HORIZON = 30
_D = 32
_HID = 32
_R = 8


def _tnt_kernel(feat_ref, cx_ref, cy_ref,
                wp1a_ref, wp1x_ref, wp1y_ref, bp1_ref, wp2t_ref, bp2_ref,
                wo1a_ref, wo1x_ref, wo1y_ref, bo1_ref, wo2x_ref, wo2y_ref,
                bo2_ref,
                wm1a_ref, wm1x_ref, wm1y_ref, bm1_ref, wm2_ref, bm2_ref,
                ws1f_ref, ws1t_ref, bs1_ref, ws2t_ref, bs2_ref,
                out_ref):
    def rb(v):
        return v.astype(jnp.bfloat16).astype(jnp.float32)

    def bdot(a, bref):
        return jnp.dot(a.astype(jnp.bfloat16), bref[...].astype(jnp.bfloat16),
                       preferred_element_type=jnp.float32)

    feat = feat_ref[...]
    cx = cx_ref[...]
    cy = cy_ref[...]

    fb = bdot(feat, wp1a_ref) + bp1_ref[...]
    h = jnp.maximum(
        fb[:, None, :]
        + rb(cx)[:, :, None] * rb(wp1x_ref[...])[None]
        + rb(cy)[:, :, None] * rb(wp1y_ref[...])[None], 0.0)
    logits = (jnp.sum(rb(h) * rb(wp2t_ref[...])[None], axis=-1)
              + bp2_ref[0, 0])

    lane = jax.lax.broadcasted_iota(jnp.int32, (_R, _NPAD), 1)
    neg = jnp.float32(-jnp.inf)
    logits = jnp.where(lane < _N, logits, neg)

    lmx = jnp.max(logits, axis=-1, keepdims=True)
    le = jnp.exp(logits - lmx)
    probs = le / jnp.sum(le, axis=-1, keepdims=True)
    probs = jnp.where(lane < _N, probs, neg)

    sel0 = jnp.zeros((_R, _M), jnp.int32)
    midx = jax.lax.broadcasted_iota(jnp.int32, (_R, _M), 1)

    def body(i, carry):
        cur, sel = carry
        mx = jnp.max(cur, axis=-1, keepdims=True)
        idx = jnp.min(jnp.where(cur == mx, lane, _NPAD), axis=-1)
        cur = jnp.where(lane == idx[:, None], neg, cur)
        sel = jnp.where(midx == i, idx[:, None], sel)
        return cur, sel

    _, sel = jax.lax.fori_loop(0, _M, body, (probs, sel0))

    oh = (sel[:, :, None] == lane[:, None, :]).astype(jnp.float32)
    sx = jnp.sum(oh * cx[:, None, :], axis=-1)
    sy = jnp.sum(oh * cy[:, None, :], axis=-1)

    fo = bdot(feat, wo1a_ref) + bo1_ref[...]
    ho = jnp.maximum(
        fo[:, None, :]
        + rb(sx)[:, :, None] * rb(wo1x_ref[...])[None]
        + rb(sy)[:, :, None] * rb(wo1y_ref[...])[None], 0.0)
    hob = rb(ho)
    tx = sx + jnp.sum(hob * rb(wo2x_ref[...])[None], axis=-1) + bo2_ref[0, 0]
    ty = sy + jnp.sum(hob * rb(wo2y_ref[...])[None], axis=-1) + bo2_ref[0, 1]

    fm = bdot(feat, wm1a_ref) + bm1_ref[...]
    hm = jnp.maximum(
        fm[:, None, :]
        + rb(tx)[:, :, None] * rb(wm1x_ref[...])[None]
        + rb(ty)[:, :, None] * rb(wm1y_ref[...])[None], 0.0)
    traj = (bdot(hm.reshape(_R * _M, _HID), wm2_ref)
            + bm2_ref[...])

    fs = bdot(feat, ws1f_ref) + bs1_ref[...]
    ts = bdot(traj, ws1t_ref)
    hs = jnp.maximum(fs[:, None, :] + ts.reshape(_R, _M, _HID), 0.0)
    s = (jnp.sum(rb(hs) * rb(ws2t_ref[...])[None], axis=-1)
         + bs2_ref[0, 0])

    sc = s / jnp.float32(0.01)
    smx = jnp.max(sc, axis=-1, keepdims=True)
    se = jnp.exp(sc - smx)
    s = se / jnp.sum(se, axis=-1, keepdims=True)

    ii = jax.lax.broadcasted_iota(jnp.int32, (_R, _M, _M), 1)
    jj = jax.lax.broadcasted_iota(jnp.int32, (_R, _M, _M), 2)
    si = s[:, :, None]
    sj = s[:, None, :]
    less = (sj < si) | ((sj == si) & (jj < ii))
    rank = jnp.sum(less.astype(jnp.int32), axis=2)

    pp = jax.lax.broadcasted_iota(jnp.int32, (_R, _M, _M), 1)
    perm = (rank[:, None, :] == pp).astype(jnp.float32)

    traj6 = traj.reshape(_R, _M, _HORIZON * 2)[:, :, :_K]
    out = jnp.sum(perm[:, :, :, None] * traj6[:, None, :, :], axis=2)
    out_ref[...] = out


def kernel(target_feat, target_candidate, Wp1, bp1, Wp2, bp2, Wo1, bo1,
           Wo2, bo2, Wm1, bm1, Wm2, bm2, Ws1, bs1, Ws2, bs2):
    b = target_feat.shape[0]
    f32 = jnp.float32

    cx = jnp.pad(target_candidate[:, :, 0], ((0, 0), (0, _NPAD - _N)))
    cy = jnp.pad(target_candidate[:, :, 1], ((0, 0), (0, _NPAD - _N)))

    args = (
        target_feat, cx, cy,
        Wp1[:_D], Wp1[_D].reshape(1, _HID), Wp1[_D + 1].reshape(1, _HID),
        bp1.reshape(1, _HID), Wp2.reshape(1, _HID), bp2.reshape(1, 1),
        Wo1[:_D], Wo1[_D].reshape(1, _HID), Wo1[_D + 1].reshape(1, _HID),
        bo1.reshape(1, _HID), Wo2[:, 0].reshape(1, _HID),
        Wo2[:, 1].reshape(1, _HID), bo2.reshape(1, 2),
        Wm1[:_D], Wm1[_D].reshape(1, _HID), Wm1[_D + 1].reshape(1, _HID),
        bm1.reshape(1, _HID), Wm2, bm2.reshape(1, _HORIZON * 2),
        Ws1[:_D], Ws1[_D:], bs1.reshape(1, _HID), Ws2.reshape(1, _HID),
        bs2.reshape(1, 1),
    )

    def bspec(a):
        shp = a.shape
        return pl.BlockSpec(shp, lambda i, _n=len(shp): (0,) * _n)

    in_specs = [
        pl.BlockSpec((_R, _D), lambda i: (i, 0)),
        pl.BlockSpec((_R, _NPAD), lambda i: (i, 0)),
        pl.BlockSpec((_R, _NPAD), lambda i: (i, 0)),
    ] + [bspec(a) for a in args[3:]]

    out = pl.pallas_call(
        _tnt_kernel,
        grid=(b // _R,),
        in_specs=in_specs,
        out_specs=pl.BlockSpec((_R, _M, _K), lambda i: (i, 0, 0)),
        out_shape=jax.ShapeDtypeStruct((b, _M, _K), f32),
        compiler_params=pltpu.CompilerParams(
            dimension_semantics=("arbitrary",)),
    )(*args)

    return out.reshape(b * _M, _K)
